# baseline (speedup 1.0000x reference)
import functools

import jax
import jax.numpy as jnp
import numpy as np
from jax import lax
from jax.experimental import pallas as pl
from jax.experimental.pallas import tpu as pltpu
from jax.experimental.pallas import tpu_sc as plsc

NC = 2
NS = 16
CB = 128
LOG2 = float(np.log(2.0))


def _sc_mesh():
    return plsc.VectorSubcoreMesh(core_axis_name="c", subcore_axis_name="s")


SCK = 16
SCKD = 8


@functools.cache
def _sc_degrees_call(E, N):
    nck_total = E // CB
    assert nck_total * CB == E
    nsck = -(-nck_total // SCKD)
    nw = NC * NS
    NPAD = -(-2 * N // 512) * 512

    def body(src_hbm, dst_hbm, out_hbm, s_src, s_dst, gs2, gd2, ones_v, zc_v,
             acc1, sem):
        c = lax.axis_index("c")
        s = lax.axis_index("s")
        w = s * NC + c

        for k in range(CB // 16):
            ones_v[pl.ds(k * 16, 16)] = jnp.full((16,), 1.0, jnp.float32)
        for k in range(zc_v.shape[0] // 16):
            zc_v[pl.ds(k * 16, 16)] = jnp.zeros((16,), jnp.float32)

        zchunks = NPAD // 512
        for k in range((zchunks + NS - 1) // NS):
            chunk = s + k * NS

            @pl.when(chunk < zchunks)
            def _():
                pltpu.sync_copy(zc_v.at[pl.ds(0, 512)],
                                acc1.at[pl.ds(chunk * 512, 512)])
        plsc.subcore_barrier()

        nG = ((nsck - 1 - w) // nw + 1).astype(jnp.int32)

        def sck_body(G, _):
            sck = w + G * nw
            pltpu.sync_copy(src_hbm.at[pl.ds(sck * SCKD, SCKD)], s_src)
            pltpu.sync_copy(dst_hbm.at[pl.ds(sck * SCKD, SCKD)], s_dst)
            for j in range(SCKD):
                for k in range(CB // 16):
                    sl = pl.ds(k * 16, 16)
                    gs2[j, sl] = s_src[j, sl] * 2
                    gd2[j, sl] = s_dst[j, sl] * 2 + 1
            for j in range(SCKD):
                valid = sck * SCKD + j < nck_total

                @pl.when(valid)
                def _(j=j):
                    pltpu.async_copy(ones_v, acc1.at[gs2.at[j]], sem,
                                     add=True)
                    pltpu.async_copy(ones_v, acc1.at[gd2.at[j]], sem,
                                     add=True)
            for j in range(SCKD):
                valid = sck * SCKD + j < nck_total

                @pl.when(valid)
                def _(j=j):
                    pltpu.make_async_copy(ones_v, acc1.at[gs2.at[j]],
                                          sem).wait()
                    pltpu.make_async_copy(ones_v, acc1.at[gd2.at[j]],
                                          sem).wait()
            return 0

        lax.fori_loop(0, nG, sck_body, 0)
        plsc.subcore_barrier()

        for cc in range(NC):
            @pl.when((s == 0) & (c == cc))
            def _(cc=cc):
                pltpu.sync_copy(acc1, out_hbm.at[pl.ds(cc * NPAD, NPAD)])

    return pl.kernel(
        body,
        out_type=jax.ShapeDtypeStruct((NC * NPAD,), jnp.float32),
        mesh=_sc_mesh(),
        scratch_types=[
            pltpu.VMEM((SCKD, CB), jnp.int32),
            pltpu.VMEM((SCKD, CB), jnp.int32),
            pltpu.VMEM((SCKD, CB), jnp.int32),
            pltpu.VMEM((SCKD, CB), jnp.int32),
            pltpu.VMEM((CB,), jnp.float32),
            pltpu.VMEM((512,), jnp.float32),
            pltpu.VMEM_SHARED((NPAD,), jnp.float32),
            pltpu.SemaphoreType.DMA,
        ],
    )


@functools.cache
def _sc_spmm_call(E, N, D, weighted):
    Dw = 128
    colsplit = (D == 2 * Dw)
    assert colsplit or D == Dw
    nck_total = E // CB
    assert nck_total * CB == E
    rpt = 624
    tail = N - rpt * NS

    nsck = -(-nck_total // SCK)

    NB = 2

    def body(x_hbm, src_hbm, dst_hbm, *rest):
        if weighted:
            (ew_hbm, out_hbm, s_src, s_dst, s_ew, rv0, rv1,
             acc, g0, g1, t0, t1) = rest
        else:
            (out_hbm, s_src, s_dst, rv0, rv1, acc, g0, g1, t0, t1) = rest
            ew_hbm = s_ew = None
        c = lax.axis_index("c")
        s = lax.axis_index("s")
        rows = (rv0, rv1)
        gsems = (g0, g1)
        ssems = (t0, t1)
        rows_v0 = rv0

        def zbody(j, _):
            rows_v0[j // (Dw // 16),
                    pl.ds((j % (Dw // 16)) * 16, 16)] = jnp.zeros((16,),
                                                                  jnp.float32)
            return 0

        lax.fori_loop(0, CB * (Dw // 16), zbody, 0)
        for off in range(0, rpt, CB):
            sz = min(CB, rpt - off)
            pltpu.sync_copy(rows_v0.at[pl.ds(0, sz)],
                            acc.at[pl.ds(s * rpt + off, sz)])

        @pl.when(s == NS - 1)
        def _():
            pltpu.sync_copy(rows_v0.at[pl.ds(0, tail)],
                            acc.at[pl.ds(rpt * NS, tail)])
        plsc.subcore_barrier()

        if colsplit:
            nper, first = NS, s
        else:
            nper, first = NS * NC, s * NC + c
        nG = ((nsck - 1 - first) // nper + 1).astype(jnp.int32)

        def stage(G1, ofs):
            sck = first + G1 * nper
            pltpu.sync_copy(src_hbm.at[pl.ds(sck * SCK, SCK)],
                            s_src.at[pl.ds(ofs, SCK)])
            pltpu.sync_copy(dst_hbm.at[pl.ds(sck * SCK, SCK)],
                            s_dst.at[pl.ds(ofs, SCK)])
            if weighted:
                pltpu.sync_copy(ew_hbm.at[pl.ds(sck * SCK, SCK)],
                                s_ew.at[pl.ds(ofs, SCK)])
            if colsplit:
                for j in range(SCK):
                    for k in range(CB // 16):
                        sl = pl.ds(k * 16, 16)
                        s_src[ofs + j, sl] = s_src[ofs + j, sl] * 2 + c

        def drain_scatter(b):
            pltpu.make_async_copy(rows[b], acc.at[s_dst.at[0]],
                                  ssems[b]).wait()

        def issue(row, b):
            pltpu.async_copy(x_hbm.at[s_src.at[row]], rows[b], gsems[b])

        def process(row, b):
            pltpu.make_async_copy(x_hbm.at[s_src.at[row]], rows[b],
                                  gsems[b]).wait()
            rows_b = rows[b]
            if weighted:
                def sbody(k, _):
                    wv = s_ew[row, pl.ds(k * 16, 16)]
                    for e2 in range(16):
                        w = wv[e2]
                        rr = k * 16 + e2
                        for k2 in range(Dw // 16):
                            sl2 = pl.ds(k2 * 16, 16)
                            rows_b[rr, sl2] = rows_b[rr, sl2] * w
                    return 0

                lax.fori_loop(0, CB // 16, sbody, 0)
            pltpu.async_copy(rows_b, acc.at[s_dst.at[row]], ssems[b],
                             add=True)

        stage(0, 0)
        issue(0, 0)

        def sck_body(G, _):
            sck = first + G * nper
            ofs = lax.rem(G, 2) * SCK
            nofs = SCK - ofs
            for j in range(SCK):
                if j + 1 < SCK:
                    @pl.when(sck * SCK + j + 1 < nck_total)
                    def _(j=j):
                        @pl.when(G * SCK + j + 1 >= NB)
                        def _():
                            drain_scatter((j + 1) % NB)
                        issue(ofs + j + 1, (j + 1) % NB)
                else:
                    @pl.when(G + 1 < nG)
                    def _():
                        stage(G + 1, nofs)
                        drain_scatter(0)
                        issue(nofs, 0)

                @pl.when(sck * SCK + j < nck_total)
                def _(j=j):
                    process(ofs + j, j % NB)
            return 0

        lax.fori_loop(0, nG, sck_body, 0)
        for b in range(NB):
            drain_scatter(b)
        plsc.subcore_barrier()
        pltpu.sync_copy(acc.at[pl.ds(s * rpt, rpt)],
                        out_hbm.at[c, pl.ds(s * rpt, rpt)])

        @pl.when(s == NS - 1)
        def _():
            pltpu.sync_copy(acc.at[pl.ds(rpt * NS, tail)],
                            out_hbm.at[c, pl.ds(rpt * NS, tail)])

    scratch = [pltpu.VMEM((2 * SCK, CB), jnp.int32)] * 2
    if weighted:
        scratch.append(pltpu.VMEM((2 * SCK, CB), jnp.float32))
    scratch += [pltpu.VMEM((CB, Dw), jnp.float32)] * 2
    scratch += [pltpu.VMEM_SHARED((N, Dw), jnp.float32)]
    scratch += [pltpu.SemaphoreType.DMA] * 4
    return pl.kernel(
        body,
        out_type=jax.ShapeDtypeStruct((NC, N, Dw), jnp.float32),
        mesh=_sc_mesh(),
        scratch_types=scratch,
    )


def _pad2d(v, nck):
    rows = -(-nck // SCK) * SCK
    pad = rows * CB - v.shape[0]
    return jnp.concatenate([v, jnp.zeros((pad,), v.dtype)]).reshape(rows, CB)


def _sc_spmm(x, E, src2, dst2, ew2=None):
    N, D = x.shape
    call = _sc_spmm_call(E, N, D, ew2 is not None)
    xin = x.reshape(2 * N, 128) if D == 256 else x
    args = (xin, src2, dst2) + (() if ew2 is None else (ew2,))
    out = call(*args)
    if D == 256:
        return jnp.concatenate([out[0], out[1]], axis=1)
    return out


def _prelu(x, a):
    return jnp.where(x >= 0, x, a * x)


def _tc_prep(deg2, feat):
    N, Din = feat.shape
    B = 2000
    grid = N // B

    def body(deg_ref, feat_ref, ins_ref, outs_ref, xs_ref):
        d = deg_ref[...]
        outd = d[0, :, 0:1] + d[1, :, 0:1]
        ind = d[0, :, 1:2] + d[1, :, 1:2]
        outs = lax.rsqrt(jnp.maximum(outd, 1.0))
        ins_ref[...] = lax.rsqrt(jnp.maximum(ind, 1.0))
        outs_ref[...] = outs
        xs_ref[...] = feat_ref[...] * outs

    return pl.pallas_call(
        body,
        grid=(grid,),
        in_specs=[
            pl.BlockSpec((2, B, 2), lambda i: (0, i, 0)),
            pl.BlockSpec((B, Din), lambda i: (i, 0)),
        ],
        out_specs=[
            pl.BlockSpec((B, 1), lambda i: (i, 0)),
            pl.BlockSpec((B, 1), lambda i: (i, 0)),
            pl.BlockSpec((B, Din), lambda i: (i, 0)),
        ],
        out_shape=[
            jax.ShapeDtypeStruct((N, 1), jnp.float32),
            jax.ShapeDtypeStruct((N, 1), jnp.float32),
            jax.ShapeDtypeStruct((N, Din), jnp.float32),
        ],
    )(deg2, feat)


def _tc_layer(agg, W, alpha, gid2, n_graphs, inscale=None, outscale=None):
    partial = agg.ndim == 3
    if partial:
        _, N, Dk = agg.shape
    else:
        N, Dk = agg.shape
    Do = W.shape[1]
    B = 2000
    grid = N // B
    a2 = alpha.reshape(1, 1)

    def body(*refs):
        idx = 0
        agg_ref = refs[idx]; idx += 1
        ins_ref = None
        outs_ref = None
        if inscale is not None:
            ins_ref = refs[idx]; idx += 1
        W_ref = refs[idx]; idx += 1
        a_ref = refs[idx]; idx += 1
        gid_ref = refs[idx]; idx += 1
        if outscale is not None:
            outs_ref = refs[idx]; idx += 1
        h_ref = refs[idx]; idx += 1
        pool_ref = refs[idx]; idx += 1
        hs_ref = refs[idx] if outscale is not None else None

        if partial:
            a3 = agg_ref[...]
            x = a3[0] + a3[1]
        else:
            x = agg_ref[...]
        if ins_ref is not None:
            x = x * ins_ref[...]
        h = jnp.dot(x, W_ref[...], preferred_element_type=jnp.float32)
        h = _prelu(h, a_ref[0, 0])
        h_ref[...] = h
        if hs_ref is not None:
            hs_ref[...] = h * outs_ref[...]
        cols = lax.broadcasted_iota(jnp.int32, (B, n_graphs), 1)
        pm = (cols == gid_ref[...]).astype(jnp.float32)
        contrib = lax.dot_general(pm, h, (((0,), (0,)), ((), ())),
                                  preferred_element_type=jnp.float32)

        @pl.when(pl.program_id(0) == 0)
        def _():
            pool_ref[...] = contrib

        @pl.when(pl.program_id(0) != 0)
        def _():
            pool_ref[...] += contrib

    if partial:
        in_specs = [pl.BlockSpec((2, B, Dk), lambda i: (0, i, 0))]
    else:
        in_specs = [pl.BlockSpec((B, Dk), lambda i: (i, 0))]
    inputs = [agg]
    if inscale is not None:
        in_specs.append(pl.BlockSpec((B, 1), lambda i: (i, 0)))
        inputs.append(inscale)
    in_specs += [
        pl.BlockSpec((Dk, Do), lambda i: (0, 0)),
        pl.BlockSpec((1, 1), lambda i: (0, 0)),
        pl.BlockSpec((B, 1), lambda i: (i, 0)),
    ]
    inputs += [W, a2, gid2]
    if outscale is not None:
        in_specs.append(pl.BlockSpec((B, 1), lambda i: (i, 0)))
        inputs.append(outscale)
    out_specs = [
        pl.BlockSpec((B, Do), lambda i: (i, 0)),
        pl.BlockSpec((n_graphs, Do), lambda i: (0, 0)),
    ]
    out_shape = [
        jax.ShapeDtypeStruct((N, Do), jnp.float32),
        jax.ShapeDtypeStruct((n_graphs, Do), jnp.float32),
    ]
    if outscale is not None:
        out_specs.append(pl.BlockSpec((B, Do), lambda i: (i, 0)))
        out_shape.append(jax.ShapeDtypeStruct((N, Do), jnp.float32))
    return pl.pallas_call(
        body,
        grid=(grid,),
        in_specs=in_specs,
        out_specs=out_specs,
        out_shape=out_shape,
    )(*inputs)


def _tc_mlp_global(hg, p):
    G, Di = hg.shape
    Do = p['W2'].shape[0]

    def body(x_ref, W1, b1, a1, W2, b2, a2, W3, b3, a3, Ws, bs, out_ref):
        x = x_ref[...]
        h = _prelu(jnp.dot(x, W1[...], preferred_element_type=jnp.float32)
                   + b1[...], a1[0, 0])
        h = _prelu(jnp.dot(h, W2[...], preferred_element_type=jnp.float32)
                   + b2[...], a2[0, 0])
        h = _prelu(jnp.dot(h, W3[...], preferred_element_type=jnp.float32)
                   + b3[...], a3[0, 0])
        out_ref[...] = h + jnp.dot(x, Ws[...],
                                   preferred_element_type=jnp.float32) + bs[...]

    args = [hg,
            p['W1'], p['b1'].reshape(1, Do), p['a1'].reshape(1, 1),
            p['W2'], p['b2'].reshape(1, Do), p['a2'].reshape(1, 1),
            p['W3'], p['b3'].reshape(1, Do), p['a3'].reshape(1, 1),
            p['Ws'], p['bs'].reshape(1, Do)]
    return pl.pallas_call(
        body,
        out_shape=jax.ShapeDtypeStruct((G, Do), jnp.float32),
    )(*args)


def _tc_loss(h11, h21, g1, g2, gid2, p, n_graphs):
    N, Do = h11.shape
    B = 2000
    grid = N // B

    def mlp(x, W1, b1, a1, W2, b2, a2, W3, b3, a3, Ws, bs):
        h = _prelu(jnp.dot(x, W1[...], preferred_element_type=jnp.float32)
                   + b1[...], a1[0, 0])
        h = _prelu(jnp.dot(h, W2[...], preferred_element_type=jnp.float32)
                   + b2[...], a2[0, 0])
        h = _prelu(jnp.dot(h, W3[...], preferred_element_type=jnp.float32)
                   + b3[...], a3[0, 0])
        return h + jnp.dot(x, Ws[...],
                           preferred_element_type=jnp.float32) + bs[...]

    def softplus(z):
        return jnp.maximum(z, 0.0) + jnp.log1p(jnp.exp(-jnp.abs(z)))

    def body(h11_ref, h21_ref, g1_ref, g2_ref, gid_ref,
             W1, b1, a1, W2, b2, a2, W3, b3, a3, Ws, bs, out_ref):
        mlp_args = (W1, b1, a1, W2, b2, a2, W3, b3, a3, Ws, bs)
        l1 = mlp(h11_ref[...], *mlp_args)
        l2 = mlp(h21_ref[...], *mlp_args)
        cols = lax.broadcasted_iota(jnp.int32, (B, n_graphs), 1)
        pos = (cols == gid_ref[...]).astype(jnp.float32)
        neg = 1.0 - pos
        total = jnp.float32(0.0)
        for l, g in ((l1, g2_ref), (l2, g1_ref)):
            res = lax.dot_general(l, g[...], (((1,), (1,)), ((), ())),
                                  preferred_element_type=jnp.float32)
            e_pos = (pos * (LOG2 - softplus(-res))).sum()
            e_neg = (neg * (softplus(-res) + res - LOG2)).sum()
            total += e_neg / (N * (n_graphs - 1)) - e_pos / N

        t2 = jnp.reshape(total, (1, 1))

        @pl.when(pl.program_id(0) == 0)
        def _():
            out_ref[...] = t2

        @pl.when(pl.program_id(0) != 0)
        def _():
            out_ref[...] += t2

    Dg = g1.shape[1]
    in_specs = [
        pl.BlockSpec((B, Do), lambda i: (i, 0)),
        pl.BlockSpec((B, Do), lambda i: (i, 0)),
        pl.BlockSpec((n_graphs, Dg), lambda i: (0, 0)),
        pl.BlockSpec((n_graphs, Dg), lambda i: (0, 0)),
        pl.BlockSpec((B, 1), lambda i: (i, 0)),
    ]
    args = [h11, h21, g1, g2, gid2]
    Do2 = p['W2'].shape[0]
    wlist = [p['W1'], p['b1'].reshape(1, Do2), p['a1'].reshape(1, 1),
             p['W2'], p['b2'].reshape(1, Do2), p['a2'].reshape(1, 1),
             p['W3'], p['b3'].reshape(1, Do2), p['a3'].reshape(1, 1),
             p['Ws'], p['bs'].reshape(1, Do2)]
    for wa in wlist:
        in_specs.append(pl.BlockSpec(wa.shape,
                                     functools.partial(
                                         lambda nd, i: tuple(0 for _ in
                                                             range(nd)),
                                         wa.ndim)))
        args.append(wa)
    out = pl.pallas_call(
        body,
        grid=(grid,),
        in_specs=in_specs,
        out_specs=pl.BlockSpec((1, 1), lambda i: (0, 0)),
        out_shape=jax.ShapeDtypeStruct((1, 1), jnp.float32),
    )(*args)
    return out[0, 0]


def kernel(feat, edge_index1, edge_index2, edge_weight, graph_id, params):
    N, Din = feat.shape
    E = edge_index1.shape[1]
    n_graphs = 200
    nck = E // CB
    sA2 = _pad2d(edge_index1[0], nck)
    dA2 = _pad2d(edge_index1[1], nck)
    sB2 = _pad2d(edge_index2[0], nck)
    dB2 = _pad2d(edge_index2[1], nck)
    ew2 = _pad2d(edge_weight, nck)
    gid2 = graph_id.reshape(N, 1)

    npad = -(-2 * N // 512) * 512
    deg = _sc_degrees_call(E, N)(sA2, dA2)
    deg2 = deg.reshape(2, npad)[:, :2 * N].reshape(2, N, 2)
    inscale, outscale, x1s = _tc_prep(deg2, feat)
    p1 = params['enc1']
    agg10 = _sc_spmm(x1s, E, sA2, dA2)
    h10, pool10, h10s = _tc_layer(agg10, p1['W0'], p1['a0'], gid2, n_graphs,
                                  inscale=inscale, outscale=outscale)
    agg11 = _sc_spmm(h10s, E, sA2, dA2)
    h11, pool11 = _tc_layer(agg11, p1['W1'], p1['a1'], gid2, n_graphs,
                            inscale=inscale)

    p2 = params['enc2']
    feat_dep = feat + 0.0 * h11[:, :Din]
    agg20 = _sc_spmm(feat_dep, E, sB2, dB2, ew2)
    h20, pool20 = _tc_layer(agg20, p2['W0'], p2['a0'], gid2, n_graphs)
    agg21 = _sc_spmm(h20, E, sB2, dB2, ew2)
    h21, pool21 = _tc_layer(agg21, p2['W1'], p2['a1'], gid2, n_graphs)

    hg1 = jnp.concatenate([pool10, pool11], axis=-1)
    hg2 = jnp.concatenate([pool20, pool21], axis=-1)
    g1 = _tc_mlp_global(hg1, params['global_mlp'])
    g2 = _tc_mlp_global(hg2, params['global_mlp'])

    return _tc_loss(h11, h21, g1, g2, gid2, params['local_mlp'], n_graphs)


# speedup vs baseline: 1.0811x; 1.0811x over previous
import functools

import jax
import jax.numpy as jnp
import numpy as np
from jax import lax
from jax.experimental import pallas as pl
from jax.experimental.pallas import tpu as pltpu
from jax.experimental.pallas import tpu_sc as plsc

NC = 2
NS = 16
CB = 128
LOG2 = float(np.log(2.0))


def _sc_mesh():
    return plsc.VectorSubcoreMesh(core_axis_name="c", subcore_axis_name="s")


SCK = 16
SCKD = 8


@functools.cache
def _sc_degrees_call(E, N):
    nck_total = E // CB
    assert nck_total * CB == E
    nsck = -(-nck_total // SCKD)
    nw = NC * NS
    NPAD = -(-2 * N // 512) * 512

    def body(src_hbm, dst_hbm, out_hbm, s_src, s_dst, gs2, gd2, ones_v, zc_v,
             acc1, sem):
        c = lax.axis_index("c")
        s = lax.axis_index("s")
        w = s * NC + c

        for k in range(CB // 16):
            ones_v[pl.ds(k * 16, 16)] = jnp.full((16,), 1.0, jnp.float32)
        for k in range(zc_v.shape[0] // 16):
            zc_v[pl.ds(k * 16, 16)] = jnp.zeros((16,), jnp.float32)

        zchunks = NPAD // 512
        for k in range((zchunks + NS - 1) // NS):
            chunk = s + k * NS

            @pl.when(chunk < zchunks)
            def _():
                pltpu.sync_copy(zc_v.at[pl.ds(0, 512)],
                                acc1.at[pl.ds(chunk * 512, 512)])
        plsc.subcore_barrier()

        nG = ((nsck - 1 - w) // nw + 1).astype(jnp.int32)

        def sck_body(G, _):
            sck = w + G * nw
            pltpu.sync_copy(src_hbm.at[pl.ds(sck * SCKD, SCKD)], s_src)
            pltpu.sync_copy(dst_hbm.at[pl.ds(sck * SCKD, SCKD)], s_dst)
            for j in range(SCKD):
                for k in range(CB // 16):
                    sl = pl.ds(k * 16, 16)
                    gs2[j, sl] = s_src[j, sl] * 2
                    gd2[j, sl] = s_dst[j, sl] * 2 + 1
            for j in range(SCKD):
                valid = sck * SCKD + j < nck_total

                @pl.when(valid)
                def _(j=j):
                    pltpu.async_copy(ones_v, acc1.at[gs2.at[j]], sem,
                                     add=True)
                    pltpu.async_copy(ones_v, acc1.at[gd2.at[j]], sem,
                                     add=True)
            for j in range(SCKD):
                valid = sck * SCKD + j < nck_total

                @pl.when(valid)
                def _(j=j):
                    pltpu.make_async_copy(ones_v, acc1.at[gs2.at[j]],
                                          sem).wait()
                    pltpu.make_async_copy(ones_v, acc1.at[gd2.at[j]],
                                          sem).wait()
            return 0

        lax.fori_loop(0, nG, sck_body, 0)
        plsc.subcore_barrier()

        for cc in range(NC):
            @pl.when((s == 0) & (c == cc))
            def _(cc=cc):
                pltpu.sync_copy(acc1, out_hbm.at[pl.ds(cc * NPAD, NPAD)])

    return pl.kernel(
        body,
        out_type=jax.ShapeDtypeStruct((NC * NPAD,), jnp.float32),
        mesh=_sc_mesh(),
        scratch_types=[
            pltpu.VMEM((SCKD, CB), jnp.int32),
            pltpu.VMEM((SCKD, CB), jnp.int32),
            pltpu.VMEM((SCKD, CB), jnp.int32),
            pltpu.VMEM((SCKD, CB), jnp.int32),
            pltpu.VMEM((CB,), jnp.float32),
            pltpu.VMEM((512,), jnp.float32),
            pltpu.VMEM_SHARED((NPAD,), jnp.float32),
            pltpu.SemaphoreType.DMA,
        ],
    )


@functools.cache
def _sc_spmm_call(E, N, D, weighted):
    Dw = 128
    colsplit = (D == 2 * Dw)
    assert colsplit or D == Dw
    nck_total = E // CB
    assert nck_total * CB == E
    rpt = 624
    tail = N - rpt * NS

    nsck = -(-nck_total // SCK)

    NB = 2

    def body(x_hbm, src_hbm, dst_hbm, *rest):
        if weighted:
            (ew_hbm, out_hbm, s_src, s_dst, s_ew, rv0, rv1,
             acc, g0, g1, t0, t1) = rest
        else:
            (out_hbm, s_src, s_dst, rv0, rv1, acc, g0, g1, t0, t1) = rest
            ew_hbm = s_ew = None
        c = lax.axis_index("c")
        s = lax.axis_index("s")
        rows = (rv0, rv1)
        gsems = (g0, g1)
        ssems = (t0, t1)
        rows_v0 = rv0

        def zbody(j, _):
            rows_v0[j // (Dw // 16),
                    pl.ds((j % (Dw // 16)) * 16, 16)] = jnp.zeros((16,),
                                                                  jnp.float32)
            return 0

        lax.fori_loop(0, CB * (Dw // 16), zbody, 0)
        for off in range(0, rpt, CB):
            sz = min(CB, rpt - off)
            pltpu.sync_copy(rows_v0.at[pl.ds(0, sz)],
                            acc.at[pl.ds(s * rpt + off, sz)])

        @pl.when(s == NS - 1)
        def _():
            pltpu.sync_copy(rows_v0.at[pl.ds(0, tail)],
                            acc.at[pl.ds(rpt * NS, tail)])
        plsc.subcore_barrier()

        if colsplit:
            nper, first = NS, s
        else:
            nper, first = NS * NC, s * NC + c
        nG = ((nsck - 1 - first) // nper + 1).astype(jnp.int32)

        def stage(G1, ofs):
            sck = first + G1 * nper
            pltpu.sync_copy(src_hbm.at[pl.ds(sck * SCK, SCK)],
                            s_src.at[pl.ds(ofs, SCK)])
            pltpu.sync_copy(dst_hbm.at[pl.ds(sck * SCK, SCK)],
                            s_dst.at[pl.ds(ofs, SCK)])
            if weighted:
                pltpu.sync_copy(ew_hbm.at[pl.ds(sck * SCK, SCK)],
                                s_ew.at[pl.ds(ofs, SCK)])
            if colsplit:
                for j in range(SCK):
                    for k in range(CB // 16):
                        sl = pl.ds(k * 16, 16)
                        s_src[ofs + j, sl] = s_src[ofs + j, sl] * 2 + c

        def drain_scatter(b):
            pltpu.make_async_copy(rows[b], acc.at[s_dst.at[0]],
                                  ssems[b]).wait()

        def issue(row, b):
            pltpu.async_copy(x_hbm.at[s_src.at[row]], rows[b], gsems[b])

        def process(row, b):
            pltpu.make_async_copy(x_hbm.at[s_src.at[row]], rows[b],
                                  gsems[b]).wait()
            rows_b = rows[b]
            if weighted:
                def sbody(k, _):
                    wv = s_ew[row, pl.ds(k * 16, 16)]
                    for e2 in range(16):
                        w = wv[e2]
                        rr = k * 16 + e2
                        for k2 in range(Dw // 16):
                            sl2 = pl.ds(k2 * 16, 16)
                            rows_b[rr, sl2] = rows_b[rr, sl2] * w
                    return 0

                lax.fori_loop(0, CB // 16, sbody, 0)
            pltpu.async_copy(rows_b, acc.at[s_dst.at[row]], ssems[b],
                             add=True)

        stage(0, 0)
        issue(0, 0)

        def sck_body(G, _):
            sck = first + G * nper
            ofs = lax.rem(G, 2) * SCK
            nofs = SCK - ofs
            for j in range(SCK):
                if j + 1 < SCK:
                    @pl.when(sck * SCK + j + 1 < nck_total)
                    def _(j=j):
                        @pl.when(G * SCK + j + 1 >= NB)
                        def _():
                            drain_scatter((j + 1) % NB)
                        issue(ofs + j + 1, (j + 1) % NB)
                else:
                    @pl.when(G + 1 < nG)
                    def _():
                        stage(G + 1, nofs)
                        drain_scatter(0)
                        issue(nofs, 0)

                @pl.when(sck * SCK + j < nck_total)
                def _(j=j):
                    process(ofs + j, j % NB)
            return 0

        lax.fori_loop(0, nG, sck_body, 0)
        for b in range(NB):
            drain_scatter(b)
        plsc.subcore_barrier()
        pltpu.sync_copy(acc.at[pl.ds(s * rpt, rpt)],
                        out_hbm.at[c, pl.ds(s * rpt, rpt)])

        @pl.when(s == NS - 1)
        def _():
            pltpu.sync_copy(acc.at[pl.ds(rpt * NS, tail)],
                            out_hbm.at[c, pl.ds(rpt * NS, tail)])

    scratch = [pltpu.VMEM((2 * SCK, CB), jnp.int32)] * 2
    if weighted:
        scratch.append(pltpu.VMEM((2 * SCK, CB), jnp.float32))
    scratch += [pltpu.VMEM((CB, Dw), jnp.float32)] * 2
    scratch += [pltpu.VMEM_SHARED((N, Dw), jnp.float32)]
    scratch += [pltpu.SemaphoreType.DMA] * 4
    return pl.kernel(
        body,
        out_type=jax.ShapeDtypeStruct((NC, N, Dw), jnp.float32),
        mesh=_sc_mesh(),
        scratch_types=scratch,
    )


def _pad2d(v, nck):
    rows = -(-nck // SCK) * SCK
    pad = rows * CB - v.shape[0]
    return jnp.concatenate([v, jnp.zeros((pad,), v.dtype)]).reshape(rows, CB)


def _sc_spmm(x, E, src2, dst2, ew2=None):
    N, D = x.shape
    call = _sc_spmm_call(E, N, D, ew2 is not None)
    xin = x.reshape(2 * N, 128) if D == 256 else x
    args = (xin, src2, dst2) + (() if ew2 is None else (ew2,))
    out = call(*args)
    if D == 256:
        return jnp.concatenate([out[0], out[1]], axis=1)
    return out


def _prelu(x, a):
    return jnp.where(x >= 0, x, a * x)


def _tc_prep(deg2, feat):
    N, Din = feat.shape
    B = 2000
    grid = N // B

    def body(deg_ref, feat_ref, ins_ref, outs_ref, xs_ref):
        d = deg_ref[...]
        outd = d[0, :, 0:1] + d[1, :, 0:1]
        ind = d[0, :, 1:2] + d[1, :, 1:2]
        outs = lax.rsqrt(jnp.maximum(outd, 1.0))
        ins_ref[...] = lax.rsqrt(jnp.maximum(ind, 1.0))
        outs_ref[...] = outs
        xs_ref[...] = feat_ref[...] * outs

    return pl.pallas_call(
        body,
        grid=(grid,),
        in_specs=[
            pl.BlockSpec((2, B, 2), lambda i: (0, i, 0)),
            pl.BlockSpec((B, Din), lambda i: (i, 0)),
        ],
        out_specs=[
            pl.BlockSpec((B, 1), lambda i: (i, 0)),
            pl.BlockSpec((B, 1), lambda i: (i, 0)),
            pl.BlockSpec((B, Din), lambda i: (i, 0)),
        ],
        out_shape=[
            jax.ShapeDtypeStruct((N, 1), jnp.float32),
            jax.ShapeDtypeStruct((N, 1), jnp.float32),
            jax.ShapeDtypeStruct((N, Din), jnp.float32),
        ],
    )(deg2, feat)


def _tc_layer(agg, W, alpha, gid2, n_graphs, inscale=None, outscale=None):
    partial = agg.ndim == 3
    if partial:
        _, N, Dk = agg.shape
    else:
        N, Dk = agg.shape
    Do = W.shape[1]
    B = 2000
    grid = N // B
    a2 = alpha.reshape(1, 1)

    def body(*refs):
        idx = 0
        agg_ref = refs[idx]; idx += 1
        ins_ref = None
        outs_ref = None
        if inscale is not None:
            ins_ref = refs[idx]; idx += 1
        W_ref = refs[idx]; idx += 1
        a_ref = refs[idx]; idx += 1
        gid_ref = refs[idx]; idx += 1
        if outscale is not None:
            outs_ref = refs[idx]; idx += 1
        h_ref = refs[idx]; idx += 1
        pool_ref = refs[idx]; idx += 1
        hs_ref = refs[idx] if outscale is not None else None

        if partial:
            a3 = agg_ref[...]
            x = a3[0] + a3[1]
        else:
            x = agg_ref[...]
        if ins_ref is not None:
            x = x * ins_ref[...]
        h = jnp.dot(x, W_ref[...], preferred_element_type=jnp.float32)
        h = _prelu(h, a_ref[0, 0])
        h_ref[...] = h
        if hs_ref is not None:
            hs_ref[...] = h * outs_ref[...]
        cols = lax.broadcasted_iota(jnp.int32, (B, n_graphs), 1)
        pm = (cols == gid_ref[...]).astype(jnp.float32)
        contrib = lax.dot_general(pm, h, (((0,), (0,)), ((), ())),
                                  preferred_element_type=jnp.float32)

        @pl.when(pl.program_id(0) == 0)
        def _():
            pool_ref[...] = contrib

        @pl.when(pl.program_id(0) != 0)
        def _():
            pool_ref[...] += contrib

    if partial:
        in_specs = [pl.BlockSpec((2, B, Dk), lambda i: (0, i, 0))]
    else:
        in_specs = [pl.BlockSpec((B, Dk), lambda i: (i, 0))]
    inputs = [agg]
    if inscale is not None:
        in_specs.append(pl.BlockSpec((B, 1), lambda i: (i, 0)))
        inputs.append(inscale)
    in_specs += [
        pl.BlockSpec((Dk, Do), lambda i: (0, 0)),
        pl.BlockSpec((1, 1), lambda i: (0, 0)),
        pl.BlockSpec((B, 1), lambda i: (i, 0)),
    ]
    inputs += [W, a2, gid2]
    if outscale is not None:
        in_specs.append(pl.BlockSpec((B, 1), lambda i: (i, 0)))
        inputs.append(outscale)
    out_specs = [
        pl.BlockSpec((B, Do), lambda i: (i, 0)),
        pl.BlockSpec((n_graphs, Do), lambda i: (0, 0)),
    ]
    out_shape = [
        jax.ShapeDtypeStruct((N, Do), jnp.float32),
        jax.ShapeDtypeStruct((n_graphs, Do), jnp.float32),
    ]
    if outscale is not None:
        out_specs.append(pl.BlockSpec((B, Do), lambda i: (i, 0)))
        out_shape.append(jax.ShapeDtypeStruct((N, Do), jnp.float32))
    return pl.pallas_call(
        body,
        grid=(grid,),
        in_specs=in_specs,
        out_specs=out_specs,
        out_shape=out_shape,
    )(*inputs)


def _tc_mlp_global(hg, p):
    G, Di = hg.shape
    Do = p['W2'].shape[0]

    def body(x_ref, W1, b1, a1, W2, b2, a2, W3, b3, a3, Ws, bs, out_ref):
        x = x_ref[...]
        h = _prelu(jnp.dot(x, W1[...], preferred_element_type=jnp.float32)
                   + b1[...], a1[0, 0])
        h = _prelu(jnp.dot(h, W2[...], preferred_element_type=jnp.float32)
                   + b2[...], a2[0, 0])
        h = _prelu(jnp.dot(h, W3[...], preferred_element_type=jnp.float32)
                   + b3[...], a3[0, 0])
        out_ref[...] = h + jnp.dot(x, Ws[...],
                                   preferred_element_type=jnp.float32) + bs[...]

    args = [hg,
            p['W1'], p['b1'].reshape(1, Do), p['a1'].reshape(1, 1),
            p['W2'], p['b2'].reshape(1, Do), p['a2'].reshape(1, 1),
            p['W3'], p['b3'].reshape(1, Do), p['a3'].reshape(1, 1),
            p['Ws'], p['bs'].reshape(1, Do)]
    return pl.pallas_call(
        body,
        out_shape=jax.ShapeDtypeStruct((G, Do), jnp.float32),
    )(*args)


def _tc_loss(h11, h21, g1, g2, gid2, p, n_graphs):
    N, Do = h11.shape
    B = 2000
    grid = N // B

    def mlp(x, W1, b1, a1, W2, b2, a2, W3, b3, a3, Ws, bs):
        h = _prelu(jnp.dot(x, W1[...], preferred_element_type=jnp.float32)
                   + b1[...], a1[0, 0])
        h = _prelu(jnp.dot(h, W2[...], preferred_element_type=jnp.float32)
                   + b2[...], a2[0, 0])
        h = _prelu(jnp.dot(h, W3[...], preferred_element_type=jnp.float32)
                   + b3[...], a3[0, 0])
        return h + jnp.dot(x, Ws[...],
                           preferred_element_type=jnp.float32) + bs[...]

    def softplus(z):
        return jnp.maximum(z, 0.0) + jnp.log1p(jnp.exp(-jnp.abs(z)))

    def body(h11_ref, h21_ref, g1_ref, g2_ref, gid_ref,
             W1, b1, a1, W2, b2, a2, W3, b3, a3, Ws, bs, out_ref):
        mlp_args = (W1, b1, a1, W2, b2, a2, W3, b3, a3, Ws, bs)
        l1 = mlp(h11_ref[...], *mlp_args)
        l2 = mlp(h21_ref[...], *mlp_args)
        cols = lax.broadcasted_iota(jnp.int32, (B, n_graphs), 1)
        pos = (cols == gid_ref[...]).astype(jnp.float32)
        neg = 1.0 - pos
        total = jnp.float32(0.0)
        for l, g in ((l1, g2_ref), (l2, g1_ref)):
            res = lax.dot_general(l, g[...], (((1,), (1,)), ((), ())),
                                  preferred_element_type=jnp.float32)
            e_pos = (pos * (LOG2 - softplus(-res))).sum()
            e_neg = (neg * (softplus(-res) + res - LOG2)).sum()
            total += e_neg / (N * (n_graphs - 1)) - e_pos / N

        t2 = jnp.reshape(total, (1, 1))

        @pl.when(pl.program_id(0) == 0)
        def _():
            out_ref[...] = t2

        @pl.when(pl.program_id(0) != 0)
        def _():
            out_ref[...] += t2

    Dg = g1.shape[1]
    in_specs = [
        pl.BlockSpec((B, Do), lambda i: (i, 0)),
        pl.BlockSpec((B, Do), lambda i: (i, 0)),
        pl.BlockSpec((n_graphs, Dg), lambda i: (0, 0)),
        pl.BlockSpec((n_graphs, Dg), lambda i: (0, 0)),
        pl.BlockSpec((B, 1), lambda i: (i, 0)),
    ]
    args = [h11, h21, g1, g2, gid2]
    Do2 = p['W2'].shape[0]
    wlist = [p['W1'], p['b1'].reshape(1, Do2), p['a1'].reshape(1, 1),
             p['W2'], p['b2'].reshape(1, Do2), p['a2'].reshape(1, 1),
             p['W3'], p['b3'].reshape(1, Do2), p['a3'].reshape(1, 1),
             p['Ws'], p['bs'].reshape(1, Do2)]
    for wa in wlist:
        in_specs.append(pl.BlockSpec(wa.shape,
                                     functools.partial(
                                         lambda nd, i: tuple(0 for _ in
                                                             range(nd)),
                                         wa.ndim)))
        args.append(wa)
    out = pl.pallas_call(
        body,
        grid=(grid,),
        in_specs=in_specs,
        out_specs=pl.BlockSpec((1, 1), lambda i: (0, 0)),
        out_shape=jax.ShapeDtypeStruct((1, 1), jnp.float32),
    )(*args)
    return out[0, 0]


def kernel(feat, edge_index1, edge_index2, edge_weight, graph_id, params):
    N, Din = feat.shape
    E = edge_index1.shape[1]
    n_graphs = 200
    nck = E // CB
    sA2 = _pad2d(edge_index1[0], nck)
    dA2 = _pad2d(edge_index1[1], nck)
    sB2 = _pad2d(edge_index2[0], nck)
    dB2 = _pad2d(edge_index2[1], nck)
    ew2 = _pad2d(edge_weight, nck)
    gid2 = graph_id.reshape(N, 1)

    npad = -(-2 * N // 512) * 512
    deg = _sc_degrees_call(E, N)(sA2, dA2)
    deg2 = deg.reshape(2, npad)[:, :2 * N].reshape(2, N, 2)
    inscale, outscale, x1s = _tc_prep(deg2, feat)
    p1 = params['enc1']
    agg10 = _sc_spmm(x1s, E, sA2, dA2)
    h10, pool10, h10s = _tc_layer(agg10, p1['W0'], p1['a0'], gid2, n_graphs,
                                  inscale=inscale, outscale=outscale)
    agg11 = _sc_spmm(h10s, E, sA2, dA2)
    h11, pool11 = _tc_layer(agg11, p1['W1'], p1['a1'], gid2, n_graphs,
                            inscale=inscale)

    p2 = params['enc2']
    agg20 = _sc_spmm(feat, E, sB2, dB2, ew2)
    h20, pool20 = _tc_layer(agg20, p2['W0'], p2['a0'], gid2, n_graphs)
    agg21 = _sc_spmm(h20, E, sB2, dB2, ew2)
    h21, pool21 = _tc_layer(agg21, p2['W1'], p2['a1'], gid2, n_graphs)

    hg1 = jnp.concatenate([pool10, pool11], axis=-1)
    hg2 = jnp.concatenate([pool20, pool21], axis=-1)
    g1 = _tc_mlp_global(hg1, params['global_mlp'])
    g2 = _tc_mlp_global(hg2, params['global_mlp'])

    return _tc_loss(h11, h21, g1, g2, gid2, params['local_mlp'], n_graphs)


# speedup vs baseline: 1.1017x; 1.0190x over previous
import functools

import jax
import jax.numpy as jnp
import numpy as np
from jax import lax
from jax.experimental import pallas as pl
from jax.experimental.pallas import tpu as pltpu
from jax.experimental.pallas import tpu_sc as plsc

NC = 2
NS = 16
CB = 128
LOG2 = float(np.log(2.0))


def _sc_mesh():
    return plsc.VectorSubcoreMesh(core_axis_name="c", subcore_axis_name="s")


SCK = 16
SCKD = 8


@functools.cache
def _sc_degrees_call(E, N):
    nck_total = E // CB
    assert nck_total * CB == E
    nsck = -(-nck_total // SCKD)
    nw = NC * NS
    NPAD = -(-2 * N // 512) * 512

    def body(src_hbm, dst_hbm, out_hbm, s_src, s_dst, gs2, gd2, ones_v, zc_v,
             acc1, sem):
        c = lax.axis_index("c")
        s = lax.axis_index("s")
        w = s * NC + c

        for k in range(CB // 16):
            ones_v[pl.ds(k * 16, 16)] = jnp.full((16,), 1.0, jnp.float32)
        for k in range(zc_v.shape[0] // 16):
            zc_v[pl.ds(k * 16, 16)] = jnp.zeros((16,), jnp.float32)

        zchunks = NPAD // 512
        for k in range((zchunks + NS - 1) // NS):
            chunk = s + k * NS

            @pl.when(chunk < zchunks)
            def _():
                pltpu.sync_copy(zc_v.at[pl.ds(0, 512)],
                                acc1.at[pl.ds(chunk * 512, 512)])
        plsc.subcore_barrier()

        nG = ((nsck - 1 - w) // nw + 1).astype(jnp.int32)

        def sck_body(G, _):
            sck = w + G * nw
            pltpu.sync_copy(src_hbm.at[pl.ds(sck * SCKD, SCKD)], s_src)
            pltpu.sync_copy(dst_hbm.at[pl.ds(sck * SCKD, SCKD)], s_dst)
            for j in range(SCKD):
                for k in range(CB // 16):
                    sl = pl.ds(k * 16, 16)
                    gs2[j, sl] = s_src[j, sl] * 2
                    gd2[j, sl] = s_dst[j, sl] * 2 + 1
            for j in range(SCKD):
                valid = sck * SCKD + j < nck_total

                @pl.when(valid)
                def _(j=j):
                    pltpu.async_copy(ones_v, acc1.at[gs2.at[j]], sem,
                                     add=True)
                    pltpu.async_copy(ones_v, acc1.at[gd2.at[j]], sem,
                                     add=True)
            for j in range(SCKD):
                valid = sck * SCKD + j < nck_total

                @pl.when(valid)
                def _(j=j):
                    pltpu.make_async_copy(ones_v, acc1.at[gs2.at[j]],
                                          sem).wait()
                    pltpu.make_async_copy(ones_v, acc1.at[gd2.at[j]],
                                          sem).wait()
            return 0

        lax.fori_loop(0, nG, sck_body, 0)
        plsc.subcore_barrier()

        for cc in range(NC):
            @pl.when((s == 0) & (c == cc))
            def _(cc=cc):
                pltpu.sync_copy(acc1, out_hbm.at[pl.ds(cc * NPAD, NPAD)])

    return pl.kernel(
        body,
        out_type=jax.ShapeDtypeStruct((NC * NPAD,), jnp.float32),
        mesh=_sc_mesh(),
        scratch_types=[
            pltpu.VMEM((SCKD, CB), jnp.int32),
            pltpu.VMEM((SCKD, CB), jnp.int32),
            pltpu.VMEM((SCKD, CB), jnp.int32),
            pltpu.VMEM((SCKD, CB), jnp.int32),
            pltpu.VMEM((CB,), jnp.float32),
            pltpu.VMEM((512,), jnp.float32),
            pltpu.VMEM_SHARED((NPAD,), jnp.float32),
            pltpu.SemaphoreType.DMA,
        ],
    )


@functools.cache
def _sc_spmm_call(E, N, D, weighted):
    Dw = 128
    colsplit = (D == 2 * Dw)
    assert colsplit or D == Dw
    nck_total = E // CB
    assert nck_total * CB == E
    rpt = 624
    tail = N - rpt * NS

    nsck = -(-nck_total // SCK)

    NB = 2

    def body(x_hbm, src_hbm, dst_hbm, *rest):
        if weighted:
            (ew_hbm, out_hbm, s_src, s_dst, s_ew, rv0, rv1,
             acc, g0, g1, t0, t1) = rest
        else:
            (out_hbm, s_src, s_dst, rv0, rv1, acc, g0, g1, t0, t1) = rest
            ew_hbm = s_ew = None
        c = lax.axis_index("c")
        s = lax.axis_index("s")
        rows = (rv0, rv1)
        gsems = (g0, g1)
        ssems = (t0, t1)
        rows_v0 = rv0

        def zbody(j, _):
            rows_v0[j // (Dw // 16),
                    pl.ds((j % (Dw // 16)) * 16, 16)] = jnp.zeros((16,),
                                                                  jnp.float32)
            return 0

        lax.fori_loop(0, CB * (Dw // 16), zbody, 0)
        for off in range(0, rpt, CB):
            sz = min(CB, rpt - off)
            pltpu.sync_copy(rows_v0.at[pl.ds(0, sz)],
                            acc.at[pl.ds(s * rpt + off, sz)])

        @pl.when(s == NS - 1)
        def _():
            pltpu.sync_copy(rows_v0.at[pl.ds(0, tail)],
                            acc.at[pl.ds(rpt * NS, tail)])
        plsc.subcore_barrier()

        if colsplit:
            nper, first = NS, s
        else:
            nper, first = NS * NC, s * NC + c
        nG = ((nsck - 1 - first) // nper + 1).astype(jnp.int32)

        def stage(G1, ofs):
            sck = first + G1 * nper
            pltpu.sync_copy(src_hbm.at[pl.ds(sck * SCK, SCK)],
                            s_src.at[pl.ds(ofs, SCK)])
            pltpu.sync_copy(dst_hbm.at[pl.ds(sck * SCK, SCK)],
                            s_dst.at[pl.ds(ofs, SCK)])
            if weighted:
                pltpu.sync_copy(ew_hbm.at[pl.ds(sck * SCK, SCK)],
                                s_ew.at[pl.ds(ofs, SCK)])
            if colsplit:
                for j in range(SCK):
                    for k in range(CB // 16):
                        sl = pl.ds(k * 16, 16)
                        s_src[ofs + j, sl] = s_src[ofs + j, sl] * 2 + c

        def drain_scatter(b):
            pltpu.make_async_copy(rows[b], acc.at[s_dst.at[0]],
                                  ssems[b]).wait()

        def issue(row, b):
            pltpu.async_copy(x_hbm.at[s_src.at[row]], rows[b], gsems[b])

        def process(row, b):
            pltpu.make_async_copy(x_hbm.at[s_src.at[row]], rows[b],
                                  gsems[b]).wait()
            rows_b = rows[b]
            if weighted:
                def sbody(k, _):
                    wv = s_ew[row, pl.ds(k * 16, 16)]
                    for e2 in range(16):
                        w = wv[e2]
                        rr = k * 16 + e2
                        for k2 in range(Dw // 16):
                            sl2 = pl.ds(k2 * 16, 16)
                            rows_b[rr, sl2] = rows_b[rr, sl2] * w
                    return 0

                lax.fori_loop(0, CB // 16, sbody, 0)
            pltpu.async_copy(rows_b, acc.at[s_dst.at[row]], ssems[b],
                             add=True)

        stage(0, 0)
        issue(0, 0)

        def sck_body(G, _):
            sck = first + G * nper
            ofs = lax.rem(G, 2) * SCK
            nofs = SCK - ofs
            for j in range(SCK):
                if j + 1 < SCK:
                    @pl.when(sck * SCK + j + 1 < nck_total)
                    def _(j=j):
                        @pl.when(G * SCK + j + 1 >= NB)
                        def _():
                            drain_scatter((j + 1) % NB)
                        issue(ofs + j + 1, (j + 1) % NB)
                else:
                    @pl.when(G + 1 < nG)
                    def _():
                        stage(G + 1, nofs)
                        drain_scatter(0)
                        issue(nofs, 0)

                @pl.when(sck * SCK + j < nck_total)
                def _(j=j):
                    process(ofs + j, j % NB)
            return 0

        lax.fori_loop(0, nG, sck_body, 0)
        for b in range(NB):
            drain_scatter(b)
        plsc.subcore_barrier()
        pltpu.sync_copy(acc.at[pl.ds(s * rpt, rpt)],
                        out_hbm.at[c, pl.ds(s * rpt, rpt)])

        @pl.when(s == NS - 1)
        def _():
            pltpu.sync_copy(acc.at[pl.ds(rpt * NS, tail)],
                            out_hbm.at[c, pl.ds(rpt * NS, tail)])

    scratch = [pltpu.VMEM((2 * SCK, CB), jnp.int32)] * 2
    if weighted:
        scratch.append(pltpu.VMEM((2 * SCK, CB), jnp.float32))
    scratch += [pltpu.VMEM((CB, Dw), jnp.float32)] * 2
    scratch += [pltpu.VMEM_SHARED((N, Dw), jnp.float32)]
    scratch += [pltpu.SemaphoreType.DMA] * 4
    return pl.kernel(
        body,
        out_type=jax.ShapeDtypeStruct((NC, N, Dw), jnp.float32),
        mesh=_sc_mesh(),
        scratch_types=scratch,
    )


def _pad2d(v, nck):
    rows = -(-nck // SCK) * SCK
    pad = rows * CB - v.shape[0]
    return jnp.concatenate([v, jnp.zeros((pad,), v.dtype)]).reshape(rows, CB)


def _sc_spmm(x, E, src2, dst2, ew2=None, flat_in=False):
    if flat_in:
        N, D = x.shape[0] // 2, 256
    else:
        N, D = x.shape
    call = _sc_spmm_call(E, N, D, ew2 is not None)
    args = (x, src2, dst2) + (() if ew2 is None else (ew2,))
    return call(*args)


def _prelu(x, a):
    return jnp.where(x >= 0, x, a * x)


def _tc_prep(deg2, feat):
    N, Din = feat.shape
    B = 2000
    grid = N // B

    def body(deg_ref, feat_ref, ins_ref, outs_ref, xs_ref):
        d = deg_ref[...]
        outd = d[0, :, 0:1] + d[1, :, 0:1]
        ind = d[0, :, 1:2] + d[1, :, 1:2]
        outs = lax.rsqrt(jnp.maximum(outd, 1.0))
        ins_ref[...] = lax.rsqrt(jnp.maximum(ind, 1.0))
        outs_ref[...] = outs
        xs_ref[...] = feat_ref[...] * outs

    return pl.pallas_call(
        body,
        grid=(grid,),
        in_specs=[
            pl.BlockSpec((2, B, 2), lambda i: (0, i, 0)),
            pl.BlockSpec((B, Din), lambda i: (i, 0)),
        ],
        out_specs=[
            pl.BlockSpec((B, 1), lambda i: (i, 0)),
            pl.BlockSpec((B, 1), lambda i: (i, 0)),
            pl.BlockSpec((B, Din), lambda i: (i, 0)),
        ],
        out_shape=[
            jax.ShapeDtypeStruct((N, 1), jnp.float32),
            jax.ShapeDtypeStruct((N, 1), jnp.float32),
            jax.ShapeDtypeStruct((N, Din), jnp.float32),
        ],
    )(deg2, feat)


def _tc_layer(agg, W, alpha, gid2, n_graphs, inscale=None, outscale=None,
              out_flat=False):
    partial = colhalves = False
    if agg.ndim == 3:
        _, N, Dk = agg.shape
        colhalves = (W.shape[0] == 2 * Dk)
        partial = not colhalves
    else:
        N, Dk = agg.shape
    Do = W.shape[1]
    B = 2000
    grid = N // B
    a2 = alpha.reshape(1, 1)
    extra = out_flat or (outscale is not None)

    def body(*refs):
        idx = 0
        agg_ref = refs[idx]; idx += 1
        ins_ref = None
        outs_ref = None
        if inscale is not None:
            ins_ref = refs[idx]; idx += 1
        W_ref = refs[idx]; idx += 1
        a_ref = refs[idx]; idx += 1
        gid_ref = refs[idx]; idx += 1
        if outscale is not None:
            outs_ref = refs[idx]; idx += 1
        h_ref = refs[idx]; idx += 1
        pool_ref = refs[idx]; idx += 1
        hs_ref = refs[idx] if extra else None

        if colhalves:
            a3 = agg_ref[...]
            x0, x1 = a3[0], a3[1]
            if ins_ref is not None:
                sc = ins_ref[...]
                x0 = x0 * sc
                x1 = x1 * sc
            Wv = W_ref[...]
            h = (jnp.dot(x0, Wv[:Dk], preferred_element_type=jnp.float32)
                 + jnp.dot(x1, Wv[Dk:], preferred_element_type=jnp.float32))
        else:
            if partial:
                a3 = agg_ref[...]
                x = a3[0] + a3[1]
            else:
                x = agg_ref[...]
            if ins_ref is not None:
                x = x * ins_ref[...]
            h = jnp.dot(x, W_ref[...], preferred_element_type=jnp.float32)
        h = _prelu(h, a_ref[0, 0])
        h_ref[...] = h
        if hs_ref is not None:
            hs = h * outs_ref[...] if outs_ref is not None else h
            if out_flat:
                hs_ref[...] = hs.reshape(2 * B, Do // 2)
            else:
                hs_ref[...] = hs
        cols = lax.broadcasted_iota(jnp.int32, (B, n_graphs), 1)
        pm = (cols == gid_ref[...]).astype(jnp.float32)
        contrib = lax.dot_general(pm, h, (((0,), (0,)), ((), ())),
                                  preferred_element_type=jnp.float32)

        @pl.when(pl.program_id(0) == 0)
        def _():
            pool_ref[...] = contrib

        @pl.when(pl.program_id(0) != 0)
        def _():
            pool_ref[...] += contrib

    if partial or colhalves:
        in_specs = [pl.BlockSpec((2, B, Dk), lambda i: (0, i, 0))]
    else:
        in_specs = [pl.BlockSpec((B, Dk), lambda i: (i, 0))]
    inputs = [agg]
    if inscale is not None:
        in_specs.append(pl.BlockSpec((B, 1), lambda i: (i, 0)))
        inputs.append(inscale)
    in_specs += [
        pl.BlockSpec(W.shape, lambda i: (0, 0)),
        pl.BlockSpec((1, 1), lambda i: (0, 0)),
        pl.BlockSpec((B, 1), lambda i: (i, 0)),
    ]
    inputs += [W, a2, gid2]
    if outscale is not None:
        in_specs.append(pl.BlockSpec((B, 1), lambda i: (i, 0)))
        inputs.append(outscale)
    out_specs = [
        pl.BlockSpec((B, Do), lambda i: (i, 0)),
        pl.BlockSpec((n_graphs, Do), lambda i: (0, 0)),
    ]
    out_shape = [
        jax.ShapeDtypeStruct((N, Do), jnp.float32),
        jax.ShapeDtypeStruct((n_graphs, Do), jnp.float32),
    ]
    if extra:
        if out_flat:
            out_specs.append(pl.BlockSpec((2 * B, Do // 2), lambda i: (i, 0)))
            out_shape.append(jax.ShapeDtypeStruct((2 * N, Do // 2),
                                                  jnp.float32))
        else:
            out_specs.append(pl.BlockSpec((B, Do), lambda i: (i, 0)))
            out_shape.append(jax.ShapeDtypeStruct((N, Do), jnp.float32))
    return pl.pallas_call(
        body,
        grid=(grid,),
        in_specs=in_specs,
        out_specs=out_specs,
        out_shape=out_shape,
    )(*inputs)


def _tc_mlp_global(hg, p):
    G, Di = hg.shape
    Do = p['W2'].shape[0]

    def body(x_ref, W1, b1, a1, W2, b2, a2, W3, b3, a3, Ws, bs, out_ref):
        x = x_ref[...]
        h = _prelu(jnp.dot(x, W1[...], preferred_element_type=jnp.float32)
                   + b1[...], a1[0, 0])
        h = _prelu(jnp.dot(h, W2[...], preferred_element_type=jnp.float32)
                   + b2[...], a2[0, 0])
        h = _prelu(jnp.dot(h, W3[...], preferred_element_type=jnp.float32)
                   + b3[...], a3[0, 0])
        out_ref[...] = h + jnp.dot(x, Ws[...],
                                   preferred_element_type=jnp.float32) + bs[...]

    args = [hg,
            p['W1'], p['b1'].reshape(1, Do), p['a1'].reshape(1, 1),
            p['W2'], p['b2'].reshape(1, Do), p['a2'].reshape(1, 1),
            p['W3'], p['b3'].reshape(1, Do), p['a3'].reshape(1, 1),
            p['Ws'], p['bs'].reshape(1, Do)]
    return pl.pallas_call(
        body,
        out_shape=jax.ShapeDtypeStruct((G, Do), jnp.float32),
    )(*args)


def _tc_loss(h11, h21, g1, g2, gid2, p, n_graphs):
    N, Do = h11.shape
    B = 2000
    grid = N // B

    def mlp(x, W1, b1, a1, W2, b2, a2, W3, b3, a3, Ws, bs):
        h = _prelu(jnp.dot(x, W1[...], preferred_element_type=jnp.float32)
                   + b1[...], a1[0, 0])
        h = _prelu(jnp.dot(h, W2[...], preferred_element_type=jnp.float32)
                   + b2[...], a2[0, 0])
        h = _prelu(jnp.dot(h, W3[...], preferred_element_type=jnp.float32)
                   + b3[...], a3[0, 0])
        return h + jnp.dot(x, Ws[...],
                           preferred_element_type=jnp.float32) + bs[...]

    def softplus(z):
        return jnp.maximum(z, 0.0) + jnp.log1p(jnp.exp(-jnp.abs(z)))

    def body(h11_ref, h21_ref, g1_ref, g2_ref, gid_ref,
             W1, b1, a1, W2, b2, a2, W3, b3, a3, Ws, bs, out_ref):
        mlp_args = (W1, b1, a1, W2, b2, a2, W3, b3, a3, Ws, bs)
        l1 = mlp(h11_ref[...], *mlp_args)
        l2 = mlp(h21_ref[...], *mlp_args)
        cols = lax.broadcasted_iota(jnp.int32, (B, n_graphs), 1)
        pos = (cols == gid_ref[...]).astype(jnp.float32)
        neg = 1.0 - pos
        total = jnp.float32(0.0)
        for l, g in ((l1, g2_ref), (l2, g1_ref)):
            res = lax.dot_general(l, g[...], (((1,), (1,)), ((), ())),
                                  preferred_element_type=jnp.float32)
            e_pos = (pos * (LOG2 - softplus(-res))).sum()
            e_neg = (neg * (softplus(-res) + res - LOG2)).sum()
            total += e_neg / (N * (n_graphs - 1)) - e_pos / N

        t2 = jnp.reshape(total, (1, 1))

        @pl.when(pl.program_id(0) == 0)
        def _():
            out_ref[...] = t2

        @pl.when(pl.program_id(0) != 0)
        def _():
            out_ref[...] += t2

    Dg = g1.shape[1]
    in_specs = [
        pl.BlockSpec((B, Do), lambda i: (i, 0)),
        pl.BlockSpec((B, Do), lambda i: (i, 0)),
        pl.BlockSpec((n_graphs, Dg), lambda i: (0, 0)),
        pl.BlockSpec((n_graphs, Dg), lambda i: (0, 0)),
        pl.BlockSpec((B, 1), lambda i: (i, 0)),
    ]
    args = [h11, h21, g1, g2, gid2]
    Do2 = p['W2'].shape[0]
    wlist = [p['W1'], p['b1'].reshape(1, Do2), p['a1'].reshape(1, 1),
             p['W2'], p['b2'].reshape(1, Do2), p['a2'].reshape(1, 1),
             p['W3'], p['b3'].reshape(1, Do2), p['a3'].reshape(1, 1),
             p['Ws'], p['bs'].reshape(1, Do2)]
    for wa in wlist:
        in_specs.append(pl.BlockSpec(wa.shape,
                                     functools.partial(
                                         lambda nd, i: tuple(0 for _ in
                                                             range(nd)),
                                         wa.ndim)))
        args.append(wa)
    out = pl.pallas_call(
        body,
        grid=(grid,),
        in_specs=in_specs,
        out_specs=pl.BlockSpec((1, 1), lambda i: (0, 0)),
        out_shape=jax.ShapeDtypeStruct((1, 1), jnp.float32),
    )(*args)
    return out[0, 0]


def kernel(feat, edge_index1, edge_index2, edge_weight, graph_id, params):
    N, Din = feat.shape
    E = edge_index1.shape[1]
    n_graphs = 200
    nck = E // CB
    sA2 = _pad2d(edge_index1[0], nck)
    dA2 = _pad2d(edge_index1[1], nck)
    sB2 = _pad2d(edge_index2[0], nck)
    dB2 = _pad2d(edge_index2[1], nck)
    ew2 = _pad2d(edge_weight, nck)
    gid2 = graph_id.reshape(N, 1)

    npad = -(-2 * N // 512) * 512
    deg = _sc_degrees_call(E, N)(sA2, dA2)
    deg2 = deg.reshape(2, npad)[:, :2 * N].reshape(2, N, 2)
    inscale, outscale, x1s = _tc_prep(deg2, feat)
    p1 = params['enc1']
    agg10 = _sc_spmm(x1s, E, sA2, dA2)
    h10, pool10, h10s = _tc_layer(agg10, p1['W0'], p1['a0'], gid2, n_graphs,
                                  inscale=inscale, outscale=outscale,
                                  out_flat=True)
    agg11 = _sc_spmm(h10s, E, sA2, dA2, flat_in=True)
    h11, pool11 = _tc_layer(agg11, p1['W1'], p1['a1'], gid2, n_graphs,
                            inscale=inscale)

    p2 = params['enc2']
    agg20 = _sc_spmm(feat, E, sB2, dB2, ew2)
    h20, pool20, h20f = _tc_layer(agg20, p2['W0'], p2['a0'], gid2, n_graphs,
                                  out_flat=True)
    agg21 = _sc_spmm(h20f, E, sB2, dB2, ew2, flat_in=True)
    h21, pool21 = _tc_layer(agg21, p2['W1'], p2['a1'], gid2, n_graphs)

    hg1 = jnp.concatenate([pool10, pool11], axis=-1)
    hg2 = jnp.concatenate([pool20, pool21], axis=-1)
    g1 = _tc_mlp_global(hg1, params['global_mlp'])
    g2 = _tc_mlp_global(hg2, params['global_mlp'])

    return _tc_loss(h11, h21, g1, g2, gid2, params['local_mlp'], n_graphs)
